# baseline (speedup 1.0000x reference)
import functools

import jax
import jax.numpy as jnp
import numpy as np
from jax import lax
from jax.experimental import pallas as pl
from jax.experimental.pallas import tpu as pltpu
from jax.experimental.pallas import tpu_sc as plsc

_H, _W = 32, 32
_B, _D = 64, 768
_T = _H * _W
_ROWS = _B * _T

_NC, _NS = 2, 16
_NW = _NC * _NS
_ROWS_W = _ROWS // _NW
_K = _W
_NGRP = _ROWS_W // _K
_NODD = _NGRP // 2
_NEVEN = _NGRP // 2
_ESTREAM = 8
_NJ = _NODD + _ESTREAM
_NE = _NEVEN - _ESTREAM
_NB = 2
_NBE = 2


def _zigzag_order(h, w):
    order = []
    for i in range(h):
        cols = range(w) if i % 2 == 0 else range(w - 1, -1, -1)
        order.extend(i * w + j for j in cols)
    return np.array(order, dtype=np.int32)


_SRC_ODD = np.ascontiguousarray(
    ((np.arange(_B, dtype=np.int32)[:, None] * _T
      + _zigzag_order(_H, _W)[None, :])
     .reshape(_NW, _NGRP, _K))[:, 1::2, :])

_MESH = plsc.VectorSubcoreMesh(
    core_axis_name="c", subcore_axis_name="s",
    num_cores=_NC, num_subcores=_NS,
)


@functools.partial(
    pl.kernel,
    out_type=jax.ShapeDtypeStruct((_ROWS, _D), jnp.float32),
    mesh=_MESH,
    scratch_types=[
        pltpu.VMEM((_NODD, _K), jnp.int32),
        pltpu.VMEM_SHARED((_NS, _NBE, _K, _D), jnp.float32),
    ] + [pltpu.VMEM((_K, _D), jnp.float32) for _ in range(_NB)]
      + [pltpu.SemaphoreType.DMA for _ in range(2 * _NB + 2 * _NBE)],
)
def _zigzag_sc(x_hbm, idx_hbm, out_hbm, idx_v, spm, *rest):
    bufs = rest[:_NB]
    sems_in = rest[_NB:2 * _NB]
    sems_out = rest[2 * _NB:3 * _NB]
    esems_in = rest[3 * _NB:3 * _NB + _NBE]
    esems_out = rest[3 * _NB + _NBE:3 * _NB + 2 * _NBE]

    cid = lax.axis_index("c")
    sid = lax.axis_index("s")
    wid = sid * _NC + cid
    base = wid * _ROWS_W

    pltpu.sync_copy(idx_hbm.at[wid], idx_v)

    def sout_off(j):
        if j < _NODD:
            return base + (2 * j + 1) * _K
        return base + 2 * (j - _NODD) * _K

    def start_in(j, b):
        if j < _NODD:
            pltpu.make_async_copy(
                x_hbm.at[idx_v.at[j]], bufs[b], sems_in[b]).start()
        else:
            pltpu.make_async_copy(
                x_hbm.at[pl.ds(sout_off(j), _K)], bufs[b], sems_in[b]).start()

    def wait_in(b):
        pltpu.make_async_copy(x_hbm.at[idx_v.at[0]], bufs[b], sems_in[b]).wait()

    def start_out(j, b):
        pltpu.make_async_copy(
            bufs[b], out_hbm.at[pl.ds(sout_off(j), _K)], sems_out[b]).start()

    def wait_out(b):
        pltpu.make_async_copy(
            bufs[b], out_hbm.at[pl.ds(base, _K)], sems_out[b]).wait()

    def eoff(e):
        return base + 2 * (e + _ESTREAM) * _K

    def estart_in(e, b):
        pltpu.make_async_copy(
            x_hbm.at[pl.ds(eoff(e), _K)], spm.at[sid, b], esems_in[b]).start()

    def ewait_in(b):
        pltpu.make_async_copy(
            x_hbm.at[pl.ds(base, _K)], spm.at[sid, b], esems_in[b]).wait()

    def estart_out(e, b):
        pltpu.make_async_copy(
            spm.at[sid, b], out_hbm.at[pl.ds(eoff(e), _K)], esems_out[b]).start()

    def ewait_out(b):
        pltpu.make_async_copy(
            spm.at[sid, b], out_hbm.at[pl.ds(base, _K)], esems_out[b]).wait()

    def emit_stream(j):
        b = j % _NB
        wait_in(b)
        start_out(j, b)
        if j + 1 < _NJ:
            ob = (j + 1) % _NB
            if j >= 1:
                wait_out(ob)
            start_in(j + 1, ob)

    def emit_dma(e):
        b = e % _NBE
        ewait_in(b)
        estart_out(e, b)
        if e + 1 < _NE:
            ob = (e + 1) % _NBE
            if e >= 1:
                ewait_out(ob)
            estart_in(e + 1, ob)

    start_in(0, 0)
    estart_in(0, 0)
    acc = 0
    ej = 0
    for sj in range(_NJ):
        emit_stream(sj)
        acc += _NE
        while acc >= _NJ and ej < _NE:
            emit_dma(ej)
            ej += 1
            acc -= _NJ
    while ej < _NE:
        emit_dma(ej)
        ej += 1

    for b in range(_NB):
        wait_out(b)
    for b in range(_NBE):
        ewait_out(b)


def kernel(x):
    x2 = x.reshape(_ROWS, _D)
    idx = jnp.asarray(_SRC_ODD)
    out = _zigzag_sc(x2, idx)
    return out.reshape(_B, _T, _D)


# speedup vs baseline: 1.0198x; 1.0198x over previous
import functools

import jax
import jax.numpy as jnp
import numpy as np
from jax import lax
from jax.experimental import pallas as pl
from jax.experimental.pallas import tpu as pltpu
from jax.experimental.pallas import tpu_sc as plsc

_H, _W = 32, 32
_B, _D = 64, 768
_T = _H * _W
_ROWS = _B * _T

_NC, _NS = 2, 16
_NW = _NC * _NS
_ROWS_W = _ROWS // _NW
_K = _W
_NGRP = _ROWS_W // _K
_NODD = _NGRP // 2
_NEVEN = _NGRP // 2
_ESTREAM = 0
_NJ = _NODD + _ESTREAM
_NE = _NEVEN - _ESTREAM
_NB = 2
_NBE = 2


def _zigzag_order(h, w):
    order = []
    for i in range(h):
        cols = range(w) if i % 2 == 0 else range(w - 1, -1, -1)
        order.extend(i * w + j for j in cols)
    return np.array(order, dtype=np.int32)


_SRC_ODD = np.ascontiguousarray(
    ((np.arange(_B, dtype=np.int32)[:, None] * _T
      + _zigzag_order(_H, _W)[None, :])
     .reshape(_NW, _NGRP, _K))[:, 1::2, :])

_MESH = plsc.VectorSubcoreMesh(
    core_axis_name="c", subcore_axis_name="s",
    num_cores=_NC, num_subcores=_NS,
)


@functools.partial(
    pl.kernel,
    out_type=jax.ShapeDtypeStruct((_ROWS, _D), jnp.float32),
    mesh=_MESH,
    scratch_types=[
        pltpu.VMEM((_NODD, _K), jnp.int32),
        pltpu.VMEM_SHARED((_NS, _NBE, _K, _D), jnp.float32),
    ] + [pltpu.VMEM((_K, _D), jnp.float32) for _ in range(_NB)]
      + [pltpu.SemaphoreType.DMA for _ in range(2 * _NB + 2 * _NBE)],
)
def _zigzag_sc(x_hbm, idx_hbm, out_hbm, idx_v, spm, *rest):
    bufs = rest[:_NB]
    sems_in = rest[_NB:2 * _NB]
    sems_out = rest[2 * _NB:3 * _NB]
    esems_in = rest[3 * _NB:3 * _NB + _NBE]
    esems_out = rest[3 * _NB + _NBE:3 * _NB + 2 * _NBE]

    cid = lax.axis_index("c")
    sid = lax.axis_index("s")
    wid = sid * _NC + cid
    base = wid * _ROWS_W

    pltpu.sync_copy(idx_hbm.at[wid], idx_v)

    def sout_off(j):
        if j < _NODD:
            return base + (2 * j + 1) * _K
        return base + 2 * (j - _NODD) * _K

    def start_in(j, b):
        if j < _NODD:
            pltpu.make_async_copy(
                x_hbm.at[idx_v.at[j]], bufs[b], sems_in[b]).start()
        else:
            pltpu.make_async_copy(
                x_hbm.at[pl.ds(sout_off(j), _K)], bufs[b], sems_in[b]).start()

    def wait_in(b):
        pltpu.make_async_copy(x_hbm.at[idx_v.at[0]], bufs[b], sems_in[b]).wait()

    def start_out(j, b):
        pltpu.make_async_copy(
            bufs[b], out_hbm.at[pl.ds(sout_off(j), _K)], sems_out[b]).start()

    def wait_out(b):
        pltpu.make_async_copy(
            bufs[b], out_hbm.at[pl.ds(base, _K)], sems_out[b]).wait()

    def eoff(e):
        return base + 2 * (e + _ESTREAM) * _K

    def estart_in(e, b):
        pltpu.make_async_copy(
            x_hbm.at[pl.ds(eoff(e), _K)], spm.at[sid, b], esems_in[b]).start()

    def ewait_in(b):
        pltpu.make_async_copy(
            x_hbm.at[pl.ds(base, _K)], spm.at[sid, b], esems_in[b]).wait()

    def estart_out(e, b):
        pltpu.make_async_copy(
            spm.at[sid, b], out_hbm.at[pl.ds(eoff(e), _K)], esems_out[b]).start()

    def ewait_out(b):
        pltpu.make_async_copy(
            spm.at[sid, b], out_hbm.at[pl.ds(base, _K)], esems_out[b]).wait()

    def emit_stream(j):
        b = j % _NB
        wait_in(b)
        start_out(j, b)
        if j + 1 < _NJ:
            ob = (j + 1) % _NB
            if j >= 1:
                wait_out(ob)
            start_in(j + 1, ob)

    def emit_dma(e):
        b = e % _NBE
        ewait_in(b)
        estart_out(e, b)
        if e + 1 < _NE:
            ob = (e + 1) % _NBE
            if e >= 1:
                ewait_out(ob)
            estart_in(e + 1, ob)

    start_in(0, 0)
    estart_in(0, 0)
    acc = 0
    ej = 0
    for sj in range(_NJ):
        emit_stream(sj)
        acc += _NE
        while acc >= _NJ and ej < _NE:
            emit_dma(ej)
            ej += 1
            acc -= _NJ
    while ej < _NE:
        emit_dma(ej)
        ej += 1

    for b in range(_NB):
        wait_out(b)
    for b in range(_NBE):
        ewait_out(b)


def kernel(x):
    x2 = x.reshape(_ROWS, _D)
    idx = jnp.asarray(_SRC_ODD)
    out = _zigzag_sc(x2, idx)
    return out.reshape(_B, _T, _D)
